# baseline (speedup 1.0000x reference)
import jax
import jax.numpy as jnp
from jax.experimental import pallas as pl
from jax.experimental.pallas import tpu as pltpu

_D = 64


def _mlp_kernel(x_hbm, w1_ref, b1_ref, w2_ref, b2_ref, wo_ref, bo_ref,
                out_ref, x_vmem, sem):
    cp = pltpu.make_async_copy(x_hbm.at[:, 0:1, :], x_vmem, sem)
    cp.start()
    cp.wait()
    x0 = x_vmem[:, 0, :]
    old = jnp.dot(x0, w1_ref[...], preferred_element_type=jnp.float32)
    old = jnp.maximum(old + b1_ref[...], 0.0)
    new = jnp.dot(old, w2_ref[...], preferred_element_type=jnp.float32)
    new = jnp.maximum(new + b2_ref[...], 0.0)
    logits = (
        jnp.dot(old, wo_ref[:_D, :], preferred_element_type=jnp.float32)
        + jnp.dot(new, wo_ref[_D:, :], preferred_element_type=jnp.float32)
        + bo_ref[...]
    )
    m = jnp.max(logits, axis=-1, keepdims=True)
    e = jnp.exp(logits - m)
    out_ref[...] = e / jnp.sum(e, axis=-1, keepdims=True)


def kernel(x_feats, edge_index, agg_W, agg_b, W1, b1, W2, b2, W_out, b_out):
    del edge_index, agg_W, agg_b
    B, _, D = x_feats.shape
    H = W1.shape[1]
    C = W_out.shape[1]
    return pl.pallas_call(
        _mlp_kernel,
        in_specs=[
            pl.BlockSpec(memory_space=pltpu.MemorySpace.HBM),
            pl.BlockSpec(memory_space=pltpu.MemorySpace.VMEM),
            pl.BlockSpec(memory_space=pltpu.MemorySpace.VMEM),
            pl.BlockSpec(memory_space=pltpu.MemorySpace.VMEM),
            pl.BlockSpec(memory_space=pltpu.MemorySpace.VMEM),
            pl.BlockSpec(memory_space=pltpu.MemorySpace.VMEM),
            pl.BlockSpec(memory_space=pltpu.MemorySpace.VMEM),
        ],
        scratch_shapes=[
            pltpu.VMEM((B, 1, D), jnp.float32),
            pltpu.SemaphoreType.DMA,
        ],
        out_shape=jax.ShapeDtypeStruct((B, C), jnp.float32),
    )(
        x_feats,
        W1,
        b1.reshape(1, H),
        W2,
        b2.reshape(1, H),
        W_out,
        b_out.reshape(1, C),
    )


# speedup vs baseline: 1.7894x; 1.7894x over previous
import jax
import jax.numpy as jnp
from jax.experimental import pallas as pl

_D = 64
_C = 16


def _mlp_kernel(p_ref, out_ref):
    w1 = p_ref[0:_D, 0:_D]
    w2 = p_ref[0:_D, _D:2 * _D]
    wo = p_ref[_D:_D + 2 * _D, 0:_C]
    b1 = p_ref[192:193, 0:_D]
    b2 = p_ref[192:193, _D:2 * _D]
    x0 = p_ref[200:204, 0:_D]
    bo = p_ref[200:201, _D:_D + _C]
    old = jnp.dot(x0, w1, preferred_element_type=jnp.float32)
    old = jnp.maximum(old + b1, 0.0)
    new = jnp.dot(old, w2, preferred_element_type=jnp.float32)
    new = jnp.maximum(new + b2, 0.0)
    logits = (
        jnp.dot(old, wo[:_D, :], preferred_element_type=jnp.float32)
        + jnp.dot(new, wo[_D:, :], preferred_element_type=jnp.float32)
        + bo
    )
    m = jnp.max(logits, axis=-1, keepdims=True)
    e = jnp.exp(logits - m)
    out_ref[...] = e / jnp.sum(e, axis=-1, keepdims=True)


def kernel(x_feats, edge_index, agg_W, agg_b, W1, b1, W2, b2, W_out, b_out):
    del edge_index, agg_W, agg_b
    B, _, D = x_feats.shape
    C = W_out.shape[1]
    x0 = jax.lax.slice_in_dim(x_feats, 0, 1, axis=1).reshape(B, D)
    p = jnp.zeros((208, 128), jnp.float32)
    p = jax.lax.dynamic_update_slice(p, W1, (0, 0))
    p = jax.lax.dynamic_update_slice(p, W2, (0, D))
    p = jax.lax.dynamic_update_slice(p, W_out, (D, 0))
    p = jax.lax.dynamic_update_slice(p, b1.reshape(1, D), (192, 0))
    p = jax.lax.dynamic_update_slice(p, b2.reshape(1, D), (192, D))
    p = jax.lax.dynamic_update_slice(p, x0, (200, 0))
    p = jax.lax.dynamic_update_slice(p, b_out.reshape(1, C), (200, D))
    return pl.pallas_call(
        _mlp_kernel,
        out_shape=jax.ShapeDtypeStruct((B, C), jnp.float32),
    )(p)


# speedup vs baseline: 2.5344x; 1.4164x over previous
import jax
import jax.numpy as jnp
from jax.experimental import pallas as pl

_D = 64


def _mlp_kernel(x_ref, wcat_ref, wo_ref, bcat_ref, out_ref):
    x0 = x_ref[...]
    old = jnp.dot(x0, wcat_ref[:, :_D], preferred_element_type=jnp.float32)
    old = jnp.maximum(old + bcat_ref[:, 0:_D], 0.0)
    new = jnp.dot(old, wcat_ref[:, _D:], preferred_element_type=jnp.float32)
    new = jnp.maximum(new + bcat_ref[:, _D:2 * _D], 0.0)
    logits = (
        jnp.dot(old, wo_ref[:_D, :], preferred_element_type=jnp.float32)
        + jnp.dot(new, wo_ref[_D:, :], preferred_element_type=jnp.float32)
        + bcat_ref[:, 2 * _D:]
    )
    m = jnp.max(logits, axis=-1, keepdims=True)
    e = jnp.exp(logits - m)
    out_ref[...] = e / jnp.sum(e, axis=-1, keepdims=True)


def kernel(x_feats, edge_index, agg_W, agg_b, W1, b1, W2, b2, W_out, b_out):
    del edge_index, agg_W, agg_b
    B, _, D = x_feats.shape
    H = W1.shape[1]
    C = W_out.shape[1]
    x0 = jax.lax.slice_in_dim(x_feats, 0, 1, axis=1).reshape(B, D)
    wcat = jnp.concatenate([W1, W2], axis=1)
    bcat = jnp.concatenate([b1, b2, b_out]).reshape(1, -1)
    return pl.pallas_call(
        _mlp_kernel,
        out_shape=jax.ShapeDtypeStruct((B, C), jnp.float32),
    )(x0, wcat, W_out, bcat)
